# Initial kernel scaffold; baseline (speedup 1.0000x reference)
import jax
import jax.numpy as jnp
from jax.experimental import pallas as pl


def kernel(x, pos, edge_index, batch_ids, params):
    raise NotImplementedError("write your pallas kernel here")


# speedup vs baseline: 3.0540x; 3.0540x over previous
import functools

import jax
import jax.numpy as jnp
from jax import lax
from jax.experimental import pallas as pl
from jax.experimental.pallas import tpu as pltpu
from jax.experimental.pallas import tpu_sc as plsc

NC, NS, LANES = 2, 16, 16
NW = NC * NS
D = 128
G = 64
CH = 80


def _sc_mesh():
    return plsc.VectorSubcoreMesh(
        core_axis_name="c", subcore_axis_name="s",
        num_cores=NC, num_subcores=NS)


_SC_PARAMS = pltpu.CompilerParams(needs_layout_passes=False)


@functools.cache
def _build_d2(E, N):
    EP = E // NW

    @functools.partial(
        pl.kernel,
        out_type=jax.ShapeDtypeStruct((E,), jnp.float32),
        mesh=_sc_mesh(),
        compiler_params=_SC_PARAMS,
        scratch_types=[
            pltpu.VMEM((N,), jnp.float32),
            pltpu.VMEM((N,), jnp.float32),
            pltpu.VMEM((N,), jnp.float32),
            pltpu.VMEM((EP,), jnp.int32),
            pltpu.VMEM((EP,), jnp.int32),
            pltpu.VMEM((EP,), jnp.float32),
        ],
    )
    def k(px_hbm, py_hbm, pz_hbm, src_hbm, dst_hbm, d2_hbm,
          px_v, py_v, pz_v, src_v, dst_v, out_v):
        wid = lax.axis_index("s") * NC + lax.axis_index("c")
        base = wid * EP
        pltpu.sync_copy(px_hbm, px_v)
        pltpu.sync_copy(py_hbm, py_v)
        pltpu.sync_copy(pz_hbm, pz_v)
        pltpu.sync_copy(src_hbm.at[pl.ds(base, EP)], src_v)
        pltpu.sync_copy(dst_hbm.at[pl.ds(base, EP)], dst_v)

        def body(i, carry):
            s16 = src_v[pl.ds(i * LANES, LANES)]
            d16 = dst_v[pl.ds(i * LANES, LANES)]
            acc = jnp.zeros((LANES,), jnp.float32)
            for pref in (px_v, py_v, pz_v):
                a = plsc.load_gather(pref, [d16])
                b = plsc.load_gather(pref, [s16])
                df = a - b
                acc = acc + df * df
            out_v[pl.ds(i * LANES, LANES)] = acc
            return carry

        lax.fori_loop(0, EP // LANES, body, 0)
        pltpu.sync_copy(out_v, d2_hbm.at[pl.ds(base, EP)])

    return k


@functools.cache
def _build_gather(E, N):
    EP = E // NW
    nchunk = EP // CH

    @functools.partial(
        pl.kernel,
        out_type=(jax.ShapeDtypeStruct((E, D), jnp.float32),
                  jax.ShapeDtypeStruct((E, D), jnp.float32)),
        mesh=_sc_mesh(),
        compiler_params=_SC_PARAMS,
        scratch_types=[
            pltpu.VMEM((EP,), jnp.int32),
            pltpu.VMEM((EP,), jnp.int32),
            pltpu.VMEM((CH, D), jnp.float32),
            pltpu.VMEM((CH, D), jnp.float32),
            pltpu.SemaphoreType.DMA,
            pltpu.SemaphoreType.DMA,
        ],
    )
    def k(A_hbm, B_hbm, src_hbm, dst_hbm, outA_hbm, outB_hbm,
          src_v, dst_v, bufA, bufB, semA, semB):
        wid = lax.axis_index("s") * NC + lax.axis_index("c")
        base = wid * EP
        pltpu.sync_copy(src_hbm.at[pl.ds(base, EP)], src_v)
        pltpu.sync_copy(dst_hbm.at[pl.ds(base, EP)], dst_v)

        def body(kk, carry):
            off = kk * CH
            cpA = pltpu.async_copy(A_hbm.at[dst_v.at[pl.ds(off, CH)]], bufA, semA)
            cpB = pltpu.async_copy(B_hbm.at[src_v.at[pl.ds(off, CH)]], bufB, semB)
            cpA.wait()
            cpB.wait()
            pltpu.sync_copy(bufA, outA_hbm.at[pl.ds(base + off, CH), :])
            pltpu.sync_copy(bufB, outB_hbm.at[pl.ds(base + off, CH), :])
            return carry

        lax.fori_loop(0, nchunk, body, 0)

    return k


@functools.cache
def _build_scatter(E, NP):
    EP = E // NW
    nchunk = EP // CH
    rows_per_tile = NP // NS
    zch = 128

    @functools.partial(
        pl.kernel,
        out_type=jax.ShapeDtypeStruct((NC, NP, D), jnp.float32),
        mesh=_sc_mesh(),
        compiler_params=_SC_PARAMS,
        scratch_types=[
            pltpu.VMEM((nchunk, CH), jnp.int32),
            pltpu.VMEM((CH, D), jnp.float32),
            pltpu.VMEM((zch, D), jnp.float32),
            pltpu.VMEM_SHARED((NP, D), jnp.float32),
        ],
    )
    def k(m2_hbm, dst3_hbm, out_hbm, idx_v, buf, zbuf, acc):
        cid = lax.axis_index("c")
        sid = lax.axis_index("s")
        wid = sid * NC + cid
        base = wid * EP

        def zrow(i, carry):
            for j in range(D // LANES):
                zbuf[i, pl.ds(j * LANES, LANES)] = jnp.zeros((LANES,), jnp.float32)
            return carry

        lax.fori_loop(0, zch, zrow, 0)
        for st in range(rows_per_tile // zch):
            pltpu.sync_copy(zbuf, acc.at[pl.ds(sid * rows_per_tile + st * zch, zch), :])
        plsc.subcore_barrier()

        pltpu.sync_copy(dst3_hbm.at[wid], idx_v)

        def body(kk, carry):
            pltpu.sync_copy(m2_hbm.at[pl.ds(base + kk * CH, CH), :], buf)
            pltpu.sync_copy(buf, acc.at[idx_v.at[kk]], add=True)
            return carry

        lax.fori_loop(0, nchunk, body, 0)
        plsc.subcore_barrier()

        for st in range(rows_per_tile // zch):
            r0 = sid * rows_per_tile + st * zch
            pltpu.sync_copy(acc.at[pl.ds(r0, zch), :], zbuf)
            pltpu.sync_copy(zbuf, out_hbm.at[cid, pl.ds(r0, zch), :])

    return k


_PREC = jax.lax.Precision.DEFAULT


def _ln(v, g, b):
    mu = jnp.mean(v, axis=-1, keepdims=True)
    c = v - mu
    var = jnp.mean(c * c, axis=-1, keepdims=True)
    return c * jax.lax.rsqrt(var + 1e-5) * g + b


def _mm(a, w):
    return jnp.dot(a, w, preferred_element_type=jnp.float32, precision=_PREC)


def _full(shape):
    return pl.BlockSpec(shape, lambda i: tuple(0 for _ in shape))


@functools.cache
def _build_emb_ab(N):
    BN = 1000

    def body(x_ref, embW, embb, Wd, Ws, b0, h_ref, A_ref, B_ref):
        h = _mm(x_ref[:], embW[:]) + embb[:]
        h_ref[:] = h
        A_ref[:] = _mm(h, Wd[:]) + b0[:]
        B_ref[:] = _mm(h, Ws[:])

    row = pl.BlockSpec((BN, D), lambda i: (i, 0))
    return pl.pallas_call(
        body,
        grid=(N // BN,),
        in_specs=[row, _full((D, D)), _full((1, D)),
                  _full((D, D)), _full((D, D)), _full((1, D))],
        out_specs=(row, row, row),
        out_shape=(jax.ShapeDtypeStruct((N, D), jnp.float32),) * 3,
    )


@functools.cache
def _build_edge(E):
    BE = 512

    def body(mA, mB, d2, wc, g0, b0, m1W, m1b, g1, b1, out):
        m = mA[:] + mB[:]
        dist = jnp.sqrt(d2[:])
        m = m + dist[:, None] * wc[:]
        m = jax.nn.relu(_ln(m, g0[:], b0[:]))
        m = _mm(m, m1W[:]) + m1b[:]
        out[:] = jax.nn.relu(_ln(m, g1[:], b1[:]))

    row = pl.BlockSpec((BE, D), lambda i: (i, 0))
    vec = pl.BlockSpec((BE,), lambda i: (i,))
    return pl.pallas_call(
        body,
        grid=(E // BE,),
        in_specs=[row, row, vec, _full((1, D)), _full((1, D)), _full((1, D)),
                  _full((D, D)), _full((1, D)), _full((1, D)), _full((1, D))],
        out_specs=row,
        out_shape=jax.ShapeDtypeStruct((E, D), jnp.float32),
    )


def _update(h, parts, Uh, Um, ub, g0, b0, u1W, u1b, g1, b1):
    agg = parts[0] + parts[1]
    u = _mm(h, Uh) + _mm(agg, Um) + ub
    u = jax.nn.relu(_ln(u, g0, b0))
    u = _mm(u, u1W) + u1b
    u = jax.nn.relu(_ln(u, g1, b1))
    return h + u


@functools.cache
def _build_node_ab(N):
    BN = 1000

    def body(h_ref, p_ref, Uh, Um, ub, g0, b0, u1W, u1b, g1, b1,
             Wd, Ws, b0n, h_out, A_ref, B_ref):
        hn = _update(h_ref[:], p_ref[:], Uh[:], Um[:], ub[:], g0[:], b0[:],
                     u1W[:], u1b[:], g1[:], b1[:])
        h_out[:] = hn
        A_ref[:] = _mm(hn, Wd[:]) + b0n[:]
        B_ref[:] = _mm(hn, Ws[:])

    row = pl.BlockSpec((BN, D), lambda i: (i, 0))
    prow = pl.BlockSpec((NC, BN, D), lambda i: (0, i, 0))
    return pl.pallas_call(
        body,
        grid=(N // BN,),
        in_specs=[row, prow, _full((D, D)), _full((D, D)), _full((1, D)),
                  _full((1, D)), _full((1, D)), _full((D, D)), _full((1, D)),
                  _full((1, D)), _full((1, D)),
                  _full((D, D)), _full((D, D)), _full((1, D))],
        out_specs=(row, row, row),
        out_shape=(jax.ShapeDtypeStruct((N, D), jnp.float32),) * 3,
    )


@functools.cache
def _build_node_pool(N):
    BN = 1000
    nblk = N // BN

    def body(h_ref, p_ref, Uh, Um, ub, g0, b0, u1W, u1b, g1, b1,
             bids_ref, p0W, p0b, p1Wt, p1b, out_ref, acc):
        i = pl.program_id(0)
        hn = _update(h_ref[:], p_ref[:], Uh[:], Um[:], ub[:], g0[:], b0[:],
                     u1W[:], u1b[:], g1[:], b1[:])
        ohT = (lax.broadcasted_iota(jnp.int32, (G, BN), 0)
               == jnp.broadcast_to(bids_ref[0], (G, BN))).astype(jnp.float32)
        part = _mm(ohT, hn)

        @pl.when(i == 0)
        def _():
            acc[:] = part

        @pl.when(i > 0)
        def _():
            acc[:] = acc[:] + part

        @pl.when(i == nblk - 1)
        def _():
            t = jax.nn.relu(_mm(acc[:], p0W[:]) + p0b[:])
            out_ref[:] = jnp.sum(t * p1Wt[:], axis=-1, keepdims=True) + p1b[:]

    row = pl.BlockSpec((BN, D), lambda i: (i, 0))
    prow = pl.BlockSpec((NC, BN, D), lambda i: (0, i, 0))
    vec = pl.BlockSpec((1, 1, BN), lambda i: (i, 0, 0))
    return pl.pallas_call(
        body,
        grid=(nblk,),
        in_specs=[row, prow, _full((D, D)), _full((D, D)), _full((1, D)),
                  _full((1, D)), _full((1, D)), _full((D, D)), _full((1, D)),
                  _full((1, D)), _full((1, D)),
                  vec, _full((D, D)), _full((1, D)), _full((1, D)),
                  _full((1, 1))],
        out_specs=pl.BlockSpec((G, 1), lambda i: (0, 0)),
        out_shape=jax.ShapeDtypeStruct((G, 1), jnp.float32),
        scratch_shapes=[pltpu.VMEM((G, D), jnp.float32)],
    )


def _r(v):
    return v.reshape(1, -1)


def kernel(x, pos, edge_index, batch_ids, params):
    N = x.shape[0]
    E = edge_index.shape[1]
    src = edge_index[0]
    dst = edge_index[1]
    dst3 = dst.reshape(NW, E // (NW * CH), CH)
    NP = ((N + NS * 128 - 1) // (NS * 128)) * NS * 128

    px, py, pz = pos[:, 0], pos[:, 1], pos[:, 2]
    d2 = _build_d2(E, N)(px, py, pz, src, dst)

    layers = params["layers"]
    msg_splits = []
    for lp in layers:
        W0 = lp["msg0"]["W"]
        msg_splits.append((W0[:D], W0[D:2 * D], W0[2 * D:2 * D + 1],
                           _r(lp["msg0"]["b"])))

    Wd0, Ws0, _, b00 = msg_splits[0]
    h, A, B = _build_emb_ab(N)(
        x, params["emb"]["W"], _r(params["emb"]["b"]), Wd0, Ws0, b00)

    gather_k = _build_gather(E, N)
    edge_k = _build_edge(E)
    scatter_k = _build_scatter(E, NP)
    node_ab_k = _build_node_ab(N)
    node_pool_k = _build_node_pool(N)

    out = None
    for li, lp in enumerate(layers):
        _, _, wc, _ = msg_splits[li]
        mA, mB = gather_k(A, B, src, dst)
        m2 = edge_k(mA, mB, d2, wc,
                    _r(lp["msg_ln0"]["g"]), _r(lp["msg_ln0"]["b"]),
                    lp["msg1"]["W"], _r(lp["msg1"]["b"]),
                    _r(lp["msg_ln1"]["g"]), _r(lp["msg_ln1"]["b"]))
        parts = scatter_k(m2, dst3)

        U = lp["upd0"]["W"]
        upd_args = (U[:D], U[D:], _r(lp["upd0"]["b"]),
                    _r(lp["upd_ln0"]["g"]), _r(lp["upd_ln0"]["b"]),
                    lp["upd1"]["W"], _r(lp["upd1"]["b"]),
                    _r(lp["upd_ln1"]["g"]), _r(lp["upd_ln1"]["b"]))
        if li + 1 < len(layers):
            Wdn, Wsn, _, b0n = msg_splits[li + 1]
            h, A, B = node_ab_k(h, parts, *upd_args, Wdn, Wsn, b0n)
        else:
            bids3 = batch_ids.reshape(N // 1000, 1, 1000)
            out = node_pool_k(h, parts, *upd_args, bids3,
                              params["pred0"]["W"], _r(params["pred0"]["b"]),
                              params["pred1"]["W"].T,
                              params["pred1"]["b"].reshape(1, 1))
    return out


# speedup vs baseline: 3.5740x; 1.1703x over previous
import functools

import jax
import jax.numpy as jnp
from jax import lax
from jax.experimental import pallas as pl
from jax.experimental.pallas import tpu as pltpu
from jax.experimental.pallas import tpu_sc as plsc

NC, NS, LANES = 2, 16, 16
NW = NC * NS
D = 128
G = 64
CH = 80


def _sc_mesh():
    return plsc.VectorSubcoreMesh(
        core_axis_name="c", subcore_axis_name="s",
        num_cores=NC, num_subcores=NS)


_SC_PARAMS = pltpu.CompilerParams(needs_layout_passes=False)


@functools.cache
def _build_d2(E, N):
    EP = E // NW

    @functools.partial(
        pl.kernel,
        out_type=jax.ShapeDtypeStruct((E,), jnp.float32),
        mesh=_sc_mesh(),
        compiler_params=_SC_PARAMS,
        scratch_types=[
            pltpu.VMEM((N,), jnp.float32),
            pltpu.VMEM((N,), jnp.float32),
            pltpu.VMEM((N,), jnp.float32),
            pltpu.VMEM((EP,), jnp.int32),
            pltpu.VMEM((EP,), jnp.int32),
            pltpu.VMEM((EP,), jnp.float32),
        ],
    )
    def k(px_hbm, py_hbm, pz_hbm, src_hbm, dst_hbm, d2_hbm,
          px_v, py_v, pz_v, src_v, dst_v, out_v):
        wid = lax.axis_index("s") * NC + lax.axis_index("c")
        base = wid * EP
        pltpu.sync_copy(px_hbm, px_v)
        pltpu.sync_copy(py_hbm, py_v)
        pltpu.sync_copy(pz_hbm, pz_v)
        pltpu.sync_copy(src_hbm.at[pl.ds(base, EP)], src_v)
        pltpu.sync_copy(dst_hbm.at[pl.ds(base, EP)], dst_v)

        def body(i, carry):
            s16 = src_v[pl.ds(i * LANES, LANES)]
            d16 = dst_v[pl.ds(i * LANES, LANES)]
            acc = jnp.zeros((LANES,), jnp.float32)
            for pref in (px_v, py_v, pz_v):
                a = plsc.load_gather(pref, [d16])
                b = plsc.load_gather(pref, [s16])
                df = a - b
                acc = acc + df * df
            out_v[pl.ds(i * LANES, LANES)] = acc
            return carry

        lax.fori_loop(0, EP // LANES, body, 0)
        pltpu.sync_copy(out_v, d2_hbm.at[pl.ds(base, EP)])

    return k


@functools.cache
def _build_gather(E, N):
    EP = E // NW
    nchunk = EP // CH

    @functools.partial(
        pl.kernel,
        out_type=jax.ShapeDtypeStruct((E, D), jnp.float32),
        mesh=_sc_mesh(),
        compiler_params=_SC_PARAMS,
        scratch_types=[
            pltpu.VMEM((EP,), jnp.int32),
            pltpu.VMEM((EP,), jnp.int32),
            pltpu.VMEM((CH, D), jnp.float32),
            pltpu.VMEM((CH, D), jnp.float32),
            pltpu.VMEM((CH, D), jnp.float32),
            pltpu.VMEM((CH, D), jnp.float32),
            pltpu.SemaphoreType.DMA,
            pltpu.SemaphoreType.DMA,
            pltpu.SemaphoreType.DMA,
            pltpu.SemaphoreType.DMA,
        ],
    )
    def k(A_hbm, B_hbm, src_hbm, dst_hbm, out_hbm,
          src_v, dst_v, bA0, bB0, bA1, bB1, sA0, sB0, sA1, sB1):
        wid = lax.axis_index("s") * NC + lax.axis_index("c")
        base = wid * EP
        pltpu.sync_copy(src_hbm.at[pl.ds(base, EP)], src_v)
        pltpu.sync_copy(dst_hbm.at[pl.ds(base, EP)], dst_v)

        def start(kk, bA, bB, sA, sB):
            off = kk * CH
            pltpu.async_copy(A_hbm.at[dst_v.at[pl.ds(off, CH)]], bA, sA)
            pltpu.async_copy(B_hbm.at[src_v.at[pl.ds(off, CH)]], bB, sB)

        def finish(kk, bA, bB, sA, sB):
            pltpu.make_async_copy(A_hbm.at[dst_v.at[pl.ds(0, CH)]], bA, sA).wait()
            pltpu.make_async_copy(B_hbm.at[src_v.at[pl.ds(0, CH)]], bB, sB).wait()

            def addrow(r, carry):
                for j in range(D // LANES):
                    sl = pl.ds(j * LANES, LANES)
                    bA[r, sl] = bA[r, sl] + bB[r, sl]
                return carry

            lax.fori_loop(0, CH, addrow, 0)
            pltpu.sync_copy(bA, out_hbm.at[pl.ds(base + kk * CH, CH), :])

        start(0, bA0, bB0, sA0, sB0)
        start(1, bA1, bB1, sA1, sB1)

        def body(i, carry):
            g = i * 2
            finish(g, bA0, bB0, sA0, sB0)
            start(g + 2, bA0, bB0, sA0, sB0)

            finish(g + 1, bA1, bB1, sA1, sB1)

            @pl.when(g + 3 < nchunk)
            def _():
                start(g + 3, bA1, bB1, sA1, sB1)

            return carry

        lax.fori_loop(0, (nchunk - 1) // 2, body, 0)
        finish(nchunk - 1, bA0, bB0, sA0, sB0)

    return k


@functools.cache
def _build_scatter(E, NP):
    EP = E // NW
    nchunk = EP // CH
    rows_per_tile = NP // NS
    zch = 128

    @functools.partial(
        pl.kernel,
        out_type=jax.ShapeDtypeStruct((NC, NP, D), jnp.float32),
        mesh=_sc_mesh(),
        compiler_params=_SC_PARAMS,
        scratch_types=[
            pltpu.VMEM((nchunk, CH), jnp.int32),
            pltpu.VMEM((CH, D), jnp.float32),
            pltpu.VMEM((zch, D), jnp.float32),
            pltpu.VMEM_SHARED((NP, D), jnp.float32),
        ],
    )
    def k(m2_hbm, dst3_hbm, out_hbm, idx_v, buf, zbuf, acc):
        cid = lax.axis_index("c")
        sid = lax.axis_index("s")
        wid = sid * NC + cid
        base = wid * EP

        def zrow(i, carry):
            for j in range(D // LANES):
                zbuf[i, pl.ds(j * LANES, LANES)] = jnp.zeros((LANES,), jnp.float32)
            return carry

        lax.fori_loop(0, zch, zrow, 0)
        for st in range(rows_per_tile // zch):
            pltpu.sync_copy(zbuf, acc.at[pl.ds(sid * rows_per_tile + st * zch, zch), :])
        plsc.subcore_barrier()

        pltpu.sync_copy(dst3_hbm.at[wid], idx_v)

        def body(kk, carry):
            pltpu.sync_copy(m2_hbm.at[pl.ds(base + kk * CH, CH), :], buf)
            pltpu.sync_copy(buf, acc.at[idx_v.at[kk]], add=True)
            return carry

        lax.fori_loop(0, nchunk, body, 0)
        plsc.subcore_barrier()

        for st in range(rows_per_tile // zch):
            r0 = sid * rows_per_tile + st * zch
            pltpu.sync_copy(acc.at[pl.ds(r0, zch), :], zbuf)
            pltpu.sync_copy(zbuf, out_hbm.at[cid, pl.ds(r0, zch), :])

    return k


_PREC = jax.lax.Precision.DEFAULT


def _ln(v, g, b):
    mu = jnp.mean(v, axis=-1, keepdims=True)
    c = v - mu
    var = jnp.mean(c * c, axis=-1, keepdims=True)
    return c * jax.lax.rsqrt(var + 1e-5) * g + b


def _mm(a, w):
    return jnp.dot(a, w, preferred_element_type=jnp.float32, precision=_PREC)


def _full(shape):
    return pl.BlockSpec(shape, lambda i: tuple(0 for _ in shape))


@functools.cache
def _build_emb_ab(N):
    BN = 1000

    def body(x_ref, embW, embb, Wd, Ws, b0, h_ref, A_ref, B_ref):
        h = _mm(x_ref[:], embW[:]) + embb[:]
        h_ref[:] = h
        A_ref[:] = _mm(h, Wd[:]) + b0[:]
        B_ref[:] = _mm(h, Ws[:])

    row = pl.BlockSpec((BN, D), lambda i: (i, 0))
    return pl.pallas_call(
        body,
        grid=(N // BN,),
        in_specs=[row, _full((D, D)), _full((1, D)),
                  _full((D, D)), _full((D, D)), _full((1, D))],
        out_specs=(row, row, row),
        out_shape=(jax.ShapeDtypeStruct((N, D), jnp.float32),) * 3,
    )


@functools.cache
def _build_edge(E):
    BE = 512

    def body(m_ref, d2, wc, g0, b0, m1W, m1b, g1, b1, out):
        dist = jnp.sqrt(d2[:])
        m = m_ref[:] + dist[:, None] * wc[:]
        m = jax.nn.relu(_ln(m, g0[:], b0[:]))
        m = _mm(m, m1W[:]) + m1b[:]
        out[:] = jax.nn.relu(_ln(m, g1[:], b1[:]))

    row = pl.BlockSpec((BE, D), lambda i: (i, 0))
    vec = pl.BlockSpec((BE,), lambda i: (i,))
    return pl.pallas_call(
        body,
        grid=(E // BE,),
        in_specs=[row, vec, _full((1, D)), _full((1, D)), _full((1, D)),
                  _full((D, D)), _full((1, D)), _full((1, D)), _full((1, D))],
        out_specs=row,
        out_shape=jax.ShapeDtypeStruct((E, D), jnp.float32),
    )


def _update(h, parts, Uh, Um, ub, g0, b0, u1W, u1b, g1, b1):
    agg = parts[0] + parts[1]
    u = _mm(h, Uh) + _mm(agg, Um) + ub
    u = jax.nn.relu(_ln(u, g0, b0))
    u = _mm(u, u1W) + u1b
    u = jax.nn.relu(_ln(u, g1, b1))
    return h + u


@functools.cache
def _build_node_ab(N):
    BN = 1000

    def body(h_ref, p_ref, Uh, Um, ub, g0, b0, u1W, u1b, g1, b1,
             Wd, Ws, b0n, h_out, A_ref, B_ref):
        hn = _update(h_ref[:], p_ref[:], Uh[:], Um[:], ub[:], g0[:], b0[:],
                     u1W[:], u1b[:], g1[:], b1[:])
        h_out[:] = hn
        A_ref[:] = _mm(hn, Wd[:]) + b0n[:]
        B_ref[:] = _mm(hn, Ws[:])

    row = pl.BlockSpec((BN, D), lambda i: (i, 0))
    prow = pl.BlockSpec((NC, BN, D), lambda i: (0, i, 0))
    return pl.pallas_call(
        body,
        grid=(N // BN,),
        in_specs=[row, prow, _full((D, D)), _full((D, D)), _full((1, D)),
                  _full((1, D)), _full((1, D)), _full((D, D)), _full((1, D)),
                  _full((1, D)), _full((1, D)),
                  _full((D, D)), _full((D, D)), _full((1, D))],
        out_specs=(row, row, row),
        out_shape=(jax.ShapeDtypeStruct((N, D), jnp.float32),) * 3,
    )


@functools.cache
def _build_node_pool(N):
    BN = 1000
    nblk = N // BN

    def body(h_ref, p_ref, Uh, Um, ub, g0, b0, u1W, u1b, g1, b1,
             bids_ref, p0W, p0b, p1Wt, p1b, out_ref, acc):
        i = pl.program_id(0)
        hn = _update(h_ref[:], p_ref[:], Uh[:], Um[:], ub[:], g0[:], b0[:],
                     u1W[:], u1b[:], g1[:], b1[:])
        ohT = (lax.broadcasted_iota(jnp.int32, (G, BN), 0)
               == jnp.broadcast_to(bids_ref[0], (G, BN))).astype(jnp.float32)
        part = _mm(ohT, hn)

        @pl.when(i == 0)
        def _():
            acc[:] = part

        @pl.when(i > 0)
        def _():
            acc[:] = acc[:] + part

        @pl.when(i == nblk - 1)
        def _():
            t = jax.nn.relu(_mm(acc[:], p0W[:]) + p0b[:])
            out_ref[:] = jnp.sum(t * p1Wt[:], axis=-1, keepdims=True) + p1b[:]

    row = pl.BlockSpec((BN, D), lambda i: (i, 0))
    prow = pl.BlockSpec((NC, BN, D), lambda i: (0, i, 0))
    vec = pl.BlockSpec((1, 1, BN), lambda i: (i, 0, 0))
    return pl.pallas_call(
        body,
        grid=(nblk,),
        in_specs=[row, prow, _full((D, D)), _full((D, D)), _full((1, D)),
                  _full((1, D)), _full((1, D)), _full((D, D)), _full((1, D)),
                  _full((1, D)), _full((1, D)),
                  vec, _full((D, D)), _full((1, D)), _full((1, D)),
                  _full((1, 1))],
        out_specs=pl.BlockSpec((G, 1), lambda i: (0, 0)),
        out_shape=jax.ShapeDtypeStruct((G, 1), jnp.float32),
        scratch_shapes=[pltpu.VMEM((G, D), jnp.float32)],
    )


def _r(v):
    return v.reshape(1, -1)


def kernel(x, pos, edge_index, batch_ids, params):
    N = x.shape[0]
    E = edge_index.shape[1]
    src = edge_index[0]
    dst = edge_index[1]
    dst3 = dst.reshape(NW, E // (NW * CH), CH)
    NP = ((N + NS * 128 - 1) // (NS * 128)) * NS * 128

    px, py, pz = pos[:, 0], pos[:, 1], pos[:, 2]
    d2 = _build_d2(E, N)(px, py, pz, src, dst)

    layers = params["layers"]
    msg_splits = []
    for lp in layers:
        W0 = lp["msg0"]["W"]
        msg_splits.append((W0[:D], W0[D:2 * D], W0[2 * D:2 * D + 1],
                           _r(lp["msg0"]["b"])))

    Wd0, Ws0, _, b00 = msg_splits[0]
    h, A, B = _build_emb_ab(N)(
        x, params["emb"]["W"], _r(params["emb"]["b"]), Wd0, Ws0, b00)

    gather_k = _build_gather(E, N)
    edge_k = _build_edge(E)
    scatter_k = _build_scatter(E, NP)
    node_ab_k = _build_node_ab(N)
    node_pool_k = _build_node_pool(N)

    out = None
    for li, lp in enumerate(layers):
        _, _, wc, _ = msg_splits[li]
        m_raw = gather_k(A, B, src, dst)
        m2 = edge_k(m_raw, d2, wc,
                    _r(lp["msg_ln0"]["g"]), _r(lp["msg_ln0"]["b"]),
                    lp["msg1"]["W"], _r(lp["msg1"]["b"]),
                    _r(lp["msg_ln1"]["g"]), _r(lp["msg_ln1"]["b"]))
        parts = scatter_k(m2, dst3)

        U = lp["upd0"]["W"]
        upd_args = (U[:D], U[D:], _r(lp["upd0"]["b"]),
                    _r(lp["upd_ln0"]["g"]), _r(lp["upd_ln0"]["b"]),
                    lp["upd1"]["W"], _r(lp["upd1"]["b"]),
                    _r(lp["upd_ln1"]["g"]), _r(lp["upd_ln1"]["b"]))
        if li + 1 < len(layers):
            Wdn, Wsn, _, b0n = msg_splits[li + 1]
            h, A, B = node_ab_k(h, parts, *upd_args, Wdn, Wsn, b0n)
        else:
            bids3 = batch_ids.reshape(N // 1000, 1, 1000)
            out = node_pool_k(h, parts, *upd_args, bids3,
                              params["pred0"]["W"], _r(params["pred0"]["b"]),
                              params["pred1"]["W"].T,
                              params["pred1"]["b"].reshape(1, 1))
    return out


# speedup vs baseline: 3.9260x; 1.0985x over previous
import functools

import jax
import jax.numpy as jnp
from jax import lax
from jax.experimental import pallas as pl
from jax.experimental.pallas import tpu as pltpu
from jax.experimental.pallas import tpu_sc as plsc

NC, NS, LANES = 2, 16, 16
NW = NC * NS
D = 128
G = 64
CH = 80


def _sc_mesh():
    return plsc.VectorSubcoreMesh(
        core_axis_name="c", subcore_axis_name="s",
        num_cores=NC, num_subcores=NS)


_SC_PARAMS = pltpu.CompilerParams(needs_layout_passes=False)


@functools.cache
def _build_d2(E, N):
    EP = E // NW

    @functools.partial(
        pl.kernel,
        out_type=jax.ShapeDtypeStruct((E,), jnp.float32),
        mesh=_sc_mesh(),
        compiler_params=_SC_PARAMS,
        scratch_types=[
            pltpu.VMEM((N,), jnp.float32),
            pltpu.VMEM((N,), jnp.float32),
            pltpu.VMEM((N,), jnp.float32),
            pltpu.VMEM((EP,), jnp.int32),
            pltpu.VMEM((EP,), jnp.int32),
            pltpu.VMEM((EP,), jnp.float32),
        ],
    )
    def k(px_hbm, py_hbm, pz_hbm, src_hbm, dst_hbm, d2_hbm,
          px_v, py_v, pz_v, src_v, dst_v, out_v):
        wid = lax.axis_index("s") * NC + lax.axis_index("c")
        base = wid * EP
        pltpu.sync_copy(px_hbm, px_v)
        pltpu.sync_copy(py_hbm, py_v)
        pltpu.sync_copy(pz_hbm, pz_v)
        pltpu.sync_copy(src_hbm.at[pl.ds(base, EP)], src_v)
        pltpu.sync_copy(dst_hbm.at[pl.ds(base, EP)], dst_v)

        def body(i, carry):
            s16 = src_v[pl.ds(i * LANES, LANES)]
            d16 = dst_v[pl.ds(i * LANES, LANES)]
            acc = jnp.zeros((LANES,), jnp.float32)
            for pref in (px_v, py_v, pz_v):
                a = plsc.load_gather(pref, [d16])
                b = plsc.load_gather(pref, [s16])
                df = a - b
                acc = acc + df * df
            out_v[pl.ds(i * LANES, LANES)] = acc
            return carry

        lax.fori_loop(0, EP // LANES, body, 0)
        pltpu.sync_copy(out_v, d2_hbm.at[pl.ds(base, EP)])

    return k


@functools.cache
def _build_gather(E, N):
    EP = E // NW
    nchunk = EP // CH

    @functools.partial(
        pl.kernel,
        out_type=jax.ShapeDtypeStruct((E, D), jnp.float32),
        mesh=_sc_mesh(),
        compiler_params=_SC_PARAMS,
        scratch_types=[
            pltpu.VMEM((EP,), jnp.int32),
            pltpu.VMEM((EP,), jnp.int32),
            pltpu.VMEM((CH, D), jnp.float32),
            pltpu.VMEM((CH, D), jnp.float32),
            pltpu.VMEM((CH, D), jnp.float32),
            pltpu.VMEM((CH, D), jnp.float32),
            pltpu.SemaphoreType.DMA,
            pltpu.SemaphoreType.DMA,
            pltpu.SemaphoreType.DMA,
            pltpu.SemaphoreType.DMA,
        ],
    )
    def k(A_hbm, B_hbm, src_hbm, dst_hbm, out_hbm,
          src_v, dst_v, bA0, bB0, bA1, bB1, sA0, sB0, sA1, sB1):
        wid = lax.axis_index("s") * NC + lax.axis_index("c")
        base = wid * EP
        pltpu.sync_copy(src_hbm.at[pl.ds(base, EP)], src_v)
        pltpu.sync_copy(dst_hbm.at[pl.ds(base, EP)], dst_v)

        def start(kk, bA, bB, sA, sB):
            off = kk * CH
            pltpu.async_copy(A_hbm.at[dst_v.at[pl.ds(off, CH)]], bA, sA)
            pltpu.async_copy(B_hbm.at[src_v.at[pl.ds(off, CH)]], bB, sB)

        def finish(kk, bA, bB, sA, sB):
            pltpu.make_async_copy(A_hbm.at[dst_v.at[pl.ds(0, CH)]], bA, sA).wait()
            pltpu.make_async_copy(B_hbm.at[src_v.at[pl.ds(0, CH)]], bB, sB).wait()

            def addrow(r, carry):
                for j in range(D // LANES):
                    sl = pl.ds(j * LANES, LANES)
                    bA[r, sl] = bA[r, sl] + bB[r, sl]
                return carry

            lax.fori_loop(0, CH, addrow, 0)
            pltpu.sync_copy(bA, out_hbm.at[pl.ds(base + kk * CH, CH), :])

        start(0, bA0, bB0, sA0, sB0)
        start(1, bA1, bB1, sA1, sB1)

        def body(i, carry):
            g = i * 2
            finish(g, bA0, bB0, sA0, sB0)
            start(g + 2, bA0, bB0, sA0, sB0)

            finish(g + 1, bA1, bB1, sA1, sB1)

            @pl.when(g + 3 < nchunk)
            def _():
                start(g + 3, bA1, bB1, sA1, sB1)

            return carry

        lax.fori_loop(0, (nchunk - 1) // 2, body, 0)
        finish(nchunk - 1, bA0, bB0, sA0, sB0)

    return k


@functools.cache
def _build_scatter(E, NP):
    EP = E // NW
    nchunk = EP // CH
    rows_per_tile = NP // NS
    zch = CH

    @functools.partial(
        pl.kernel,
        out_type=jax.ShapeDtypeStruct((NC, NP, D), jnp.float32),
        mesh=_sc_mesh(),
        compiler_params=_SC_PARAMS,
        scratch_types=[
            pltpu.VMEM((nchunk, CH), jnp.int32),
            pltpu.VMEM((CH, D), jnp.float32),
            pltpu.VMEM((CH, D), jnp.float32),
            pltpu.VMEM_SHARED((NP, D), jnp.float32),
            pltpu.SemaphoreType.DMA,
            pltpu.SemaphoreType.DMA,
        ],
    )
    def k(m2_hbm, dst3_hbm, out_hbm, idx_v, b0, b1, acc, s0, s1):
        cid = lax.axis_index("c")
        sid = lax.axis_index("s")
        wid = sid * NC + cid
        base = wid * EP

        def zrow(i, carry):
            for j in range(D // LANES):
                b0[i, pl.ds(j * LANES, LANES)] = jnp.zeros((LANES,), jnp.float32)
            return carry

        lax.fori_loop(0, zch, zrow, 0)
        for st in range(rows_per_tile // zch):
            pltpu.sync_copy(b0, acc.at[pl.ds(sid * rows_per_tile + st * zch, zch), :])
        plsc.subcore_barrier()

        pltpu.sync_copy(dst3_hbm.at[wid], idx_v)

        def start(kk, buf, sem):
            pltpu.async_copy(m2_hbm.at[pl.ds(base + kk * CH, CH), :], buf, sem)

        def finish(kk, buf, sem):
            pltpu.make_async_copy(m2_hbm.at[pl.ds(base, CH), :], buf, sem).wait()
            pltpu.sync_copy(buf, acc.at[idx_v.at[kk]], add=True)

        start(0, b0, s0)
        start(1, b1, s1)

        def body(i, carry):
            g = i * 2
            finish(g, b0, s0)
            start(g + 2, b0, s0)

            finish(g + 1, b1, s1)

            @pl.when(g + 3 < nchunk)
            def _():
                start(g + 3, b1, s1)

            return carry

        lax.fori_loop(0, (nchunk - 1) // 2, body, 0)
        finish(nchunk - 1, b0, s0)
        plsc.subcore_barrier()

        for st in range(rows_per_tile // zch):
            r0 = sid * rows_per_tile + st * zch
            pltpu.sync_copy(acc.at[pl.ds(r0, zch), :], b0)
            pltpu.sync_copy(b0, out_hbm.at[cid, pl.ds(r0, zch), :])

    return k


_PREC = jax.lax.Precision.DEFAULT


def _ln(v, g, b):
    mu = jnp.mean(v, axis=-1, keepdims=True)
    c = v - mu
    var = jnp.mean(c * c, axis=-1, keepdims=True)
    return c * jax.lax.rsqrt(var + 1e-5) * g + b


def _mm(a, w):
    return jnp.dot(a, w, preferred_element_type=jnp.float32, precision=_PREC)


def _full(shape):
    return pl.BlockSpec(shape, lambda i: tuple(0 for _ in shape))


@functools.cache
def _build_emb_ab(N):
    BN = 1000

    def body(x_ref, embW, embb, Wd, Ws, b0, h_ref, A_ref, B_ref):
        h = _mm(x_ref[:], embW[:]) + embb[:]
        h_ref[:] = h
        A_ref[:] = _mm(h, Wd[:]) + b0[:]
        B_ref[:] = _mm(h, Ws[:])

    row = pl.BlockSpec((BN, D), lambda i: (i, 0))
    return pl.pallas_call(
        body,
        grid=(N // BN,),
        in_specs=[row, _full((D, D)), _full((1, D)),
                  _full((D, D)), _full((D, D)), _full((1, D))],
        out_specs=(row, row, row),
        out_shape=(jax.ShapeDtypeStruct((N, D), jnp.float32),) * 3,
    )


@functools.cache
def _build_edge(E):
    BE = 512

    def body(m_ref, d2, wc, g0, b0, m1W, m1b, g1, b1, out):
        dist = jnp.sqrt(d2[:])
        m = m_ref[:] + dist[:, None] * wc[:]
        m = jax.nn.relu(_ln(m, g0[:], b0[:]))
        m = _mm(m, m1W[:]) + m1b[:]
        out[:] = jax.nn.relu(_ln(m, g1[:], b1[:]))

    row = pl.BlockSpec((BE, D), lambda i: (i, 0))
    vec = pl.BlockSpec((BE,), lambda i: (i,))
    return pl.pallas_call(
        body,
        grid=(E // BE,),
        in_specs=[row, vec, _full((1, D)), _full((1, D)), _full((1, D)),
                  _full((D, D)), _full((1, D)), _full((1, D)), _full((1, D))],
        out_specs=row,
        out_shape=jax.ShapeDtypeStruct((E, D), jnp.float32),
    )


def _update(h, parts, Uh, Um, ub, g0, b0, u1W, u1b, g1, b1):
    agg = parts[0] + parts[1]
    u = _mm(h, Uh) + _mm(agg, Um) + ub
    u = jax.nn.relu(_ln(u, g0, b0))
    u = _mm(u, u1W) + u1b
    u = jax.nn.relu(_ln(u, g1, b1))
    return h + u


@functools.cache
def _build_node_ab(N):
    BN = 1000

    def body(h_ref, p_ref, Uh, Um, ub, g0, b0, u1W, u1b, g1, b1,
             Wd, Ws, b0n, h_out, A_ref, B_ref):
        hn = _update(h_ref[:], p_ref[:], Uh[:], Um[:], ub[:], g0[:], b0[:],
                     u1W[:], u1b[:], g1[:], b1[:])
        h_out[:] = hn
        A_ref[:] = _mm(hn, Wd[:]) + b0n[:]
        B_ref[:] = _mm(hn, Ws[:])

    row = pl.BlockSpec((BN, D), lambda i: (i, 0))
    prow = pl.BlockSpec((NC, BN, D), lambda i: (0, i, 0))
    return pl.pallas_call(
        body,
        grid=(N // BN,),
        in_specs=[row, prow, _full((D, D)), _full((D, D)), _full((1, D)),
                  _full((1, D)), _full((1, D)), _full((D, D)), _full((1, D)),
                  _full((1, D)), _full((1, D)),
                  _full((D, D)), _full((D, D)), _full((1, D))],
        out_specs=(row, row, row),
        out_shape=(jax.ShapeDtypeStruct((N, D), jnp.float32),) * 3,
    )


@functools.cache
def _build_node_pool(N):
    BN = 1000
    nblk = N // BN

    def body(h_ref, p_ref, Uh, Um, ub, g0, b0, u1W, u1b, g1, b1,
             bids_ref, p0W, p0b, p1Wt, p1b, out_ref, acc):
        i = pl.program_id(0)
        hn = _update(h_ref[:], p_ref[:], Uh[:], Um[:], ub[:], g0[:], b0[:],
                     u1W[:], u1b[:], g1[:], b1[:])
        ohT = (lax.broadcasted_iota(jnp.int32, (G, BN), 0)
               == jnp.broadcast_to(bids_ref[0], (G, BN))).astype(jnp.float32)
        part = _mm(ohT, hn)

        @pl.when(i == 0)
        def _():
            acc[:] = part

        @pl.when(i > 0)
        def _():
            acc[:] = acc[:] + part

        @pl.when(i == nblk - 1)
        def _():
            t = jax.nn.relu(_mm(acc[:], p0W[:]) + p0b[:])
            out_ref[:] = jnp.sum(t * p1Wt[:], axis=-1, keepdims=True) + p1b[:]

    row = pl.BlockSpec((BN, D), lambda i: (i, 0))
    prow = pl.BlockSpec((NC, BN, D), lambda i: (0, i, 0))
    vec = pl.BlockSpec((1, 1, BN), lambda i: (i, 0, 0))
    return pl.pallas_call(
        body,
        grid=(nblk,),
        in_specs=[row, prow, _full((D, D)), _full((D, D)), _full((1, D)),
                  _full((1, D)), _full((1, D)), _full((D, D)), _full((1, D)),
                  _full((1, D)), _full((1, D)),
                  vec, _full((D, D)), _full((1, D)), _full((1, D)),
                  _full((1, 1))],
        out_specs=pl.BlockSpec((G, 1), lambda i: (0, 0)),
        out_shape=jax.ShapeDtypeStruct((G, 1), jnp.float32),
        scratch_shapes=[pltpu.VMEM((G, D), jnp.float32)],
    )


def _r(v):
    return v.reshape(1, -1)


def kernel(x, pos, edge_index, batch_ids, params):
    N = x.shape[0]
    E = edge_index.shape[1]
    src = edge_index[0]
    dst = edge_index[1]
    dst3 = dst.reshape(NW, E // (NW * CH), CH)
    NP = ((N + NS * 128 - 1) // (NS * 128)) * NS * 128

    px, py, pz = pos[:, 0], pos[:, 1], pos[:, 2]
    d2 = _build_d2(E, N)(px, py, pz, src, dst)

    layers = params["layers"]
    msg_splits = []
    for lp in layers:
        W0 = lp["msg0"]["W"]
        msg_splits.append((W0[:D], W0[D:2 * D], W0[2 * D:2 * D + 1],
                           _r(lp["msg0"]["b"])))

    Wd0, Ws0, _, b00 = msg_splits[0]
    h, A, B = _build_emb_ab(N)(
        x, params["emb"]["W"], _r(params["emb"]["b"]), Wd0, Ws0, b00)

    gather_k = _build_gather(E, N)
    edge_k = _build_edge(E)
    scatter_k = _build_scatter(E, NP)
    node_ab_k = _build_node_ab(N)
    node_pool_k = _build_node_pool(N)

    out = None
    for li, lp in enumerate(layers):
        _, _, wc, _ = msg_splits[li]
        m_raw = gather_k(A, B, src, dst)
        m2 = edge_k(m_raw, d2, wc,
                    _r(lp["msg_ln0"]["g"]), _r(lp["msg_ln0"]["b"]),
                    lp["msg1"]["W"], _r(lp["msg1"]["b"]),
                    _r(lp["msg_ln1"]["g"]), _r(lp["msg_ln1"]["b"]))
        parts = scatter_k(m2, dst3)

        U = lp["upd0"]["W"]
        upd_args = (U[:D], U[D:], _r(lp["upd0"]["b"]),
                    _r(lp["upd_ln0"]["g"]), _r(lp["upd_ln0"]["b"]),
                    lp["upd1"]["W"], _r(lp["upd1"]["b"]),
                    _r(lp["upd_ln1"]["g"]), _r(lp["upd_ln1"]["b"]))
        if li + 1 < len(layers):
            Wdn, Wsn, _, b0n = msg_splits[li + 1]
            h, A, B = node_ab_k(h, parts, *upd_args, Wdn, Wsn, b0n)
        else:
            bids3 = batch_ids.reshape(N // 1000, 1, 1000)
            out = node_pool_k(h, parts, *upd_args, bids3,
                              params["pred0"]["W"], _r(params["pred0"]["b"]),
                              params["pred1"]["W"].T,
                              params["pred1"]["b"].reshape(1, 1))
    return out
